# baseline (speedup 1.0000x reference)
import functools

import jax
import jax.numpy as jnp
from jax import lax
from jax.experimental import pallas as pl
from jax.experimental.pallas import tpu as pltpu
from jax.experimental.pallas import tpu_sc as plsc

B = 16384
K = 32
NC = 2
NS = 16
NW = NC * NS
BPW = B // NW
CHUNK = 128
NCHUNK = BPW // CHUNK
L = 16
GROUPS = BPW // L


def _sc_body(uidx_hbm, iidx_hbm, utab_hbm, itab_hbm, out_hbm,
             uidx_v, iidx_v, ubuf, ibuf, out_v, sem):
    c = lax.axis_index("c")
    s = lax.axis_index("s")
    wid = s * NC + c
    base = wid * BPW

    pltpu.sync_copy(uidx_hbm.at[pl.ds(base, BPW)], uidx_v)
    pltpu.sync_copy(iidx_hbm.at[pl.ds(base, BPW)], iidx_v)

    def fetch(k, carry):
        for j in range(NCHUNK):
            pltpu.make_async_copy(
                utab_hbm.at[k].at[uidx_v.at[pl.ds(j * CHUNK, CHUNK)]],
                ubuf.at[pl.ds(k * BPW + j * CHUNK, CHUNK)], sem).start()
            pltpu.make_async_copy(
                itab_hbm.at[k].at[iidx_v.at[pl.ds(j * CHUNK, CHUNK)]],
                ibuf.at[pl.ds(k * BPW + j * CHUNK, CHUNK)], sem).start()
        return carry

    def drain(k, carry):
        for j in range(NCHUNK):
            pltpu.make_async_copy(
                utab_hbm.at[k].at[uidx_v.at[pl.ds(j * CHUNK, CHUNK)]],
                ubuf.at[pl.ds(k * BPW + j * CHUNK, CHUNK)], sem).wait()
            pltpu.make_async_copy(
                itab_hbm.at[k].at[iidx_v.at[pl.ds(j * CHUNK, CHUNK)]],
                ibuf.at[pl.ds(k * BPW + j * CHUNK, CHUNK)], sem).wait()
        return carry

    lax.fori_loop(0, K, fetch, 0)
    lax.fori_loop(0, K, drain, 0)

    lanes = lax.iota(jnp.int32, 16)
    zeros = jnp.zeros((L,), jnp.int32)

    def group(g, carry):
        rows0 = g * L + lanes
        acc = jnp.zeros((L,), jnp.float32)
        for k in range(K):
            rows = rows0 + k * BPW
            u = plsc.load_gather(ubuf, [rows])
            v = plsc.load_gather(ibuf, [rows])
            acc = acc + u * v
        out_v[pl.ds(g * L, L)] = acc
        return carry

    lax.fori_loop(0, GROUPS, group, 0)
    pltpu.sync_copy(out_v, out_hbm.at[pl.ds(base, BPW)])


_sc_call = functools.partial(
    pl.kernel,
    out_type=jax.ShapeDtypeStruct((B,), jnp.float32),
    mesh=plsc.VectorSubcoreMesh(core_axis_name="c", subcore_axis_name="s"),
    scratch_types=[
        pltpu.VMEM((BPW,), jnp.int32),
        pltpu.VMEM((BPW,), jnp.int32),
        pltpu.VMEM((K * BPW,), jnp.float32),
        pltpu.VMEM((K * BPW,), jnp.float32),
        pltpu.VMEM((BPW,), jnp.float32),
        pltpu.SemaphoreType.DMA,
    ],
    compiler_params=pltpu.CompilerParams(
        needs_layout_passes=False, use_tc_tiling_on_sc=False),
)(_sc_body)


def kernel(user_input, item_input, user_embedding, item_embedding):
    utab = user_embedding.T
    itab = item_embedding.T
    out = _sc_call(user_input.reshape(B), item_input.reshape(B), utab, itab)
    return out.reshape(B, 1)


# speedup vs baseline: 5.6495x; 5.6495x over previous
import functools

import jax
import jax.numpy as jnp
from jax import lax
from jax.experimental import pallas as pl
from jax.experimental.pallas import tpu as pltpu
from jax.experimental.pallas import tpu_sc as plsc

B = 16384
K = 32
NC = 2
NS = 16
NW = NC * NS
BPW = B // NW
CHUNK = 128
NCHUNK = BPW // CHUNK
L = 16
GROUPS = BPW // L


def _sc_body(uidx_hbm, iidx_hbm, utab_hbm, itab_hbm, out_hbm,
             uidx_v, iidx_v, urows_v, irows_v, out_v, sem):
    c = lax.axis_index("c")
    s = lax.axis_index("s")
    wid = s * NC + c
    base = wid * BPW

    pltpu.sync_copy(uidx_hbm.at[pl.ds(base, BPW)], uidx_v)
    pltpu.sync_copy(iidx_hbm.at[pl.ds(base, BPW)], iidx_v)

    copies = []
    for j in range(NCHUNK):
        cu = pltpu.make_async_copy(
            utab_hbm.at[uidx_v.at[pl.ds(j * CHUNK, CHUNK)]],
            urows_v.at[pl.ds(j * CHUNK, CHUNK)], sem)
        ci = pltpu.make_async_copy(
            itab_hbm.at[iidx_v.at[pl.ds(j * CHUNK, CHUNK)]],
            irows_v.at[pl.ds(j * CHUNK, CHUNK)], sem)
        cu.start()
        ci.start()
        copies.append(cu)
        copies.append(ci)
    for cp in copies:
        cp.wait()

    lanes = lax.iota(jnp.int32, 16)

    def group(g, carry):
        rows = g * L + lanes
        acc = jnp.zeros((L,), jnp.float32)
        for k in range(K):
            col = jnp.full((L,), k, jnp.int32)
            u = plsc.load_gather(urows_v, [rows, col])
            v = plsc.load_gather(irows_v, [rows, col])
            acc = acc + u * v
        out_v[pl.ds(g * L, L)] = acc
        return carry

    lax.fori_loop(0, GROUPS, group, 0)
    pltpu.sync_copy(out_v, out_hbm.at[pl.ds(base, BPW)])


_sc_call = functools.partial(
    pl.kernel,
    out_type=jax.ShapeDtypeStruct((B,), jnp.float32),
    mesh=plsc.VectorSubcoreMesh(core_axis_name="c", subcore_axis_name="s"),
    scratch_types=[
        pltpu.VMEM((BPW,), jnp.int32),
        pltpu.VMEM((BPW,), jnp.int32),
        pltpu.VMEM((BPW, K), jnp.float32),
        pltpu.VMEM((BPW, K), jnp.float32),
        pltpu.VMEM((BPW,), jnp.float32),
        pltpu.SemaphoreType.DMA,
    ],
    compiler_params=pltpu.CompilerParams(
        needs_layout_passes=False, use_tc_tiling_on_sc=False),
)(_sc_body)


def kernel(user_input, item_input, user_embedding, item_embedding):
    out = _sc_call(user_input.reshape(B), item_input.reshape(B),
                   user_embedding, item_embedding)
    return out.reshape(B, 1)
